# baseline (speedup 1.0000x reference)
import functools

import jax
import jax.numpy as jnp
from jax import lax
from jax.experimental import pallas as pl
from jax.experimental.pallas import tpu as pltpu
from jax.experimental.pallas import tpu_sc as plsc

N = 10000
E = 320000
D = 128
ED = 16
CW = 8

NC = 2
NS = 16
NW = NC * NS
EPW = E // NW

CA = 125
GA = 40
NGA = EPW // (CA * GA)

CC = 40
GC = 50
NGC = EPW // (CC * GC)

RPS = 624
TAIL = N - NS * RPS


def _sc_mesh():
    return plsc.VectorSubcoreMesh(
        core_axis_name="c", subcore_axis_name="s", num_cores=NC, num_subcores=NS
    )


def _sc_scatter_attr(edge_attr, col4a, zeros16, ones8, zeros8):
    @functools.partial(
        pl.kernel,
        out_type=(
            jax.ShapeDtypeStruct((NC, N, ED), jnp.float32),
            jax.ShapeDtypeStruct((NC, N, CW), jnp.float32),
        ),
        mesh=_sc_mesh(),
        compiler_params=pltpu.CompilerParams(use_tc_tiling_on_sc=False),
        scratch_types=[
            pltpu.VMEM((GA, CA), jnp.int32),
            pltpu.VMEM((CA, ED), jnp.float32),
            pltpu.VMEM((CA, ED), jnp.float32),
            pltpu.VMEM((CA, CW), jnp.float32),
            pltpu.VMEM((CA, CW), jnp.float32),
            pltpu.VMEM_SHARED((N, ED), jnp.float32),
            pltpu.VMEM_SHARED((N, CW), jnp.float32),
            pltpu.SemaphoreType.DMA,
            pltpu.SemaphoreType.DMA,
            pltpu.SemaphoreType.DMA,
            pltpu.SemaphoreType.DMA,
            pltpu.SemaphoreType.DMA,
        ],
    )
    def k(ea, col4_h, zeros_h, ones_h, zeros8_h, aggp, cntp,
          colv, abuf0, abuf1, onesv, zb8, acc16, acccnt,
          lsem0, lsem1, ssem0, ssem1, osem):
        cid = lax.axis_index("c")
        sid = lax.axis_index("s")
        wid = cid * NS + sid

        pltpu.sync_copy(zeros_h, abuf0)
        pltpu.sync_copy(ones_h, onesv)
        pltpu.sync_copy(zeros8_h, zb8)

        r0 = sid * RPS

        @pl.loop(0, 15)
        def _(i):
            pltpu.sync_copy(abuf0.at[pl.ds(0, 40)], acc16.at[pl.ds(r0 + i * 40, 40)])
            pltpu.sync_copy(zb8.at[pl.ds(0, 40)], acccnt.at[pl.ds(r0 + i * 40, 40)])
        pltpu.sync_copy(abuf0.at[pl.ds(0, 24)], acc16.at[pl.ds(r0 + 600, 24)])
        pltpu.sync_copy(zb8.at[pl.ds(0, 24)], acccnt.at[pl.ds(r0 + 600, 24)])

        @pl.when(sid == NS - 1)
        def _():
            pltpu.sync_copy(abuf0.at[pl.ds(0, TAIL)], acc16.at[pl.ds(NS * RPS, TAIL)])
            pltpu.sync_copy(zb8.at[pl.ds(0, TAIL)], acccnt.at[pl.ds(NS * RPS, TAIL)])

        plsc.subcore_barrier()

        base = wid * EPW

        def load(j, buf, sem):
            pltpu.async_copy(ea.at[pl.ds(base + j * CA, CA)], buf, sem)

        def wait_load(buf, sem):
            pltpu.make_async_copy(ea.at[pl.ds(0, CA)], buf, sem).wait()

        def scat(j, buf, sem):
            pltpu.async_copy(buf, acc16.at[colv.at[j]], sem, add=True)

        def wait_scat(buf, sem):
            pltpu.make_async_copy(buf, acc16.at[colv.at[0]], sem).wait()

        def ones_scat(j):
            pltpu.async_copy(onesv, acccnt.at[colv.at[j]], osem, add=True)

        def wait_ones():
            pltpu.make_async_copy(onesv, acccnt.at[colv.at[0]], osem).wait()

        for g in range(NGA):
            gbase = g * GA
            pltpu.sync_copy(col4_h.at[wid, g], colv)
            load(gbase + 0, abuf0, lsem0)
            load(gbase + 1, abuf1, lsem1)

            @pl.loop(0, GA // 2 - 1)
            def _(t):
                a = gbase + 2 * t
                wait_load(abuf0, lsem0)
                scat(2 * t, abuf0, ssem0)
                ones_scat(2 * t)
                wait_load(abuf1, lsem1)
                scat(2 * t + 1, abuf1, ssem1)
                ones_scat(2 * t + 1)
                wait_scat(abuf0, ssem0)
                load(a + 2, abuf0, lsem0)
                wait_scat(abuf1, ssem1)
                load(a + 3, abuf1, lsem1)
                wait_ones()
                wait_ones()

            wait_load(abuf0, lsem0)
            scat(GA - 2, abuf0, ssem0)
            ones_scat(GA - 2)
            wait_load(abuf1, lsem1)
            scat(GA - 1, abuf1, ssem1)
            ones_scat(GA - 1)
            wait_scat(abuf0, ssem0)
            wait_scat(abuf1, ssem1)
            wait_ones()
            wait_ones()

        plsc.subcore_barrier()

        @pl.loop(0, 15)
        def _(i):
            pltpu.sync_copy(acc16.at[pl.ds(r0 + i * 40, 40)], abuf0.at[pl.ds(0, 40)])
            pltpu.sync_copy(abuf0.at[pl.ds(0, 40)], aggp.at[cid, pl.ds(r0 + i * 40, 40)])
            pltpu.sync_copy(acccnt.at[pl.ds(r0 + i * 40, 40)], zb8.at[pl.ds(0, 40)])
            pltpu.sync_copy(zb8.at[pl.ds(0, 40)], cntp.at[cid, pl.ds(r0 + i * 40, 40)])
        pltpu.sync_copy(acc16.at[pl.ds(r0 + 600, 24)], abuf0.at[pl.ds(0, 24)])
        pltpu.sync_copy(abuf0.at[pl.ds(0, 24)], aggp.at[cid, pl.ds(r0 + 600, 24)])
        pltpu.sync_copy(acccnt.at[pl.ds(r0 + 600, 24)], zb8.at[pl.ds(0, 24)])
        pltpu.sync_copy(zb8.at[pl.ds(0, 24)], cntp.at[cid, pl.ds(r0 + 600, 24)])

        @pl.when(sid == NS - 1)
        def _():
            pltpu.sync_copy(acc16.at[pl.ds(NS * RPS, TAIL)], abuf0.at[pl.ds(0, TAIL)])
            pltpu.sync_copy(abuf0.at[pl.ds(0, TAIL)], aggp.at[cid, pl.ds(NS * RPS, TAIL)])
            pltpu.sync_copy(acccnt.at[pl.ds(NS * RPS, TAIL)], zb8.at[pl.ds(0, TAIL)])
            pltpu.sync_copy(zb8.at[pl.ds(0, TAIL)], cntp.at[cid, pl.ds(NS * RPS, TAIL)])

    return k(edge_attr, col4a, zeros16, ones8, zeros8)


def _sc_gather_scatter(y, row4, col4, zeros128):
    @functools.partial(
        pl.kernel,
        out_type=jax.ShapeDtypeStruct((NC, N, D), jnp.float32),
        mesh=_sc_mesh(),
        compiler_params=pltpu.CompilerParams(use_tc_tiling_on_sc=False),
        scratch_types=[
            pltpu.VMEM((GC, CC), jnp.int32),
            pltpu.VMEM((GC, CC), jnp.int32),
            pltpu.VMEM((CC, D), jnp.float32),
            pltpu.VMEM((CC, D), jnp.float32),
            pltpu.VMEM((CC, D), jnp.float32),
            pltpu.VMEM((CC, D), jnp.float32),
            pltpu.VMEM_SHARED((N, D), jnp.float32),
            pltpu.SemaphoreType.DMA,
            pltpu.SemaphoreType.DMA,
            pltpu.SemaphoreType.DMA,
            pltpu.SemaphoreType.DMA,
            pltpu.SemaphoreType.DMA,
            pltpu.SemaphoreType.DMA,
            pltpu.SemaphoreType.DMA,
            pltpu.SemaphoreType.DMA,
        ],
    )
    def k(y_h, row4_h, col4_h, zeros_h, sp, rowv, colv,
          gbuf0, gbuf1, gbuf2, gbuf3, acc,
          gsem0, gsem1, gsem2, gsem3, ssem0, ssem1, ssem2, ssem3):
        cid = lax.axis_index("c")
        sid = lax.axis_index("s")
        wid = cid * NS + sid

        pltpu.sync_copy(zeros_h, gbuf0)

        r0 = sid * RPS

        @pl.loop(0, 15)
        def _(i):
            pltpu.sync_copy(gbuf0.at[pl.ds(0, 40)], acc.at[pl.ds(r0 + i * 40, 40)])
        pltpu.sync_copy(gbuf0.at[pl.ds(0, 24)], acc.at[pl.ds(r0 + 600, 24)])

        @pl.when(sid == NS - 1)
        def _():
            pltpu.sync_copy(gbuf0.at[pl.ds(0, TAIL)], acc.at[pl.ds(NS * RPS, TAIL)])

        plsc.subcore_barrier()

        def gath(j, buf, sem):
            pltpu.async_copy(y_h.at[rowv.at[j]], buf, sem)

        def wait_gath(buf, sem):
            pltpu.make_async_copy(y_h.at[rowv.at[0]], buf, sem).wait()

        def scat(j, buf, sem):
            pltpu.async_copy(buf, acc.at[colv.at[j]], sem, add=True)

        def wait_scat(buf, sem):
            pltpu.make_async_copy(buf, acc.at[colv.at[0]], sem).wait()

        bufs = (gbuf0, gbuf1, gbuf2, gbuf3)
        gsems = (gsem0, gsem1, gsem2, gsem3)
        ssems = (ssem0, ssem1, ssem2, ssem3)

        for g in range(NGC):
            pltpu.sync_copy(row4_h.at[wid, g], rowv)
            pltpu.sync_copy(col4_h.at[wid, g], colv)
            for i in range(4):
                gath(i, bufs[i], gsems[i])

            @pl.loop(0, GC // 4 - 1)
            def _(q):
                for i in range(4):
                    wait_gath(bufs[i], gsems[i])
                    scat(4 * q + i, bufs[i], ssems[i])
                for i in range(4):
                    wait_scat(bufs[i], ssems[i])
                    gath(4 * q + 4 + i, bufs[i], gsems[i])

            for i in range(4):
                wait_gath(bufs[i], gsems[i])
                scat(GC - 6 + i, bufs[i], ssems[i])
            wait_scat(gbuf0, ssem0)
            gath(GC - 2, gbuf0, gsem0)
            wait_scat(gbuf1, ssem1)
            gath(GC - 1, gbuf1, gsem1)
            wait_scat(gbuf2, ssem2)
            wait_scat(gbuf3, ssem3)
            wait_gath(gbuf0, gsem0)
            scat(GC - 2, gbuf0, ssem0)
            wait_gath(gbuf1, gsem1)
            scat(GC - 1, gbuf1, ssem1)
            wait_scat(gbuf0, ssem0)
            wait_scat(gbuf1, ssem1)

        plsc.subcore_barrier()

        @pl.loop(0, 15)
        def _(i):
            pltpu.sync_copy(acc.at[pl.ds(r0 + i * 40, 40)], gbuf0.at[pl.ds(0, 40)])
            pltpu.sync_copy(gbuf0.at[pl.ds(0, 40)], sp.at[cid, pl.ds(r0 + i * 40, 40)])
        pltpu.sync_copy(acc.at[pl.ds(r0 + 600, 24)], gbuf0.at[pl.ds(0, 24)])
        pltpu.sync_copy(gbuf0.at[pl.ds(0, 24)], sp.at[cid, pl.ds(r0 + 600, 24)])

        @pl.when(sid == NS - 1)
        def _():
            pltpu.sync_copy(acc.at[pl.ds(NS * RPS, TAIL)], gbuf0.at[pl.ds(0, TAIL)])
            pltpu.sync_copy(gbuf0.at[pl.ds(0, TAIL)], sp.at[cid, pl.ds(NS * RPS, TAIL)])

    return k(y, row4, col4, zeros128)


def _tc_dense1(x, aggp, cntp, W_edge, b_edge2, W_conv, edge_attr):
    def body(x_ref, aggp_ref, cntp_ref, we_ref, be_ref, wc_ref, ea_ref,
             y_ref, dinv_ref):
        agg16 = aggp_ref[0] + aggp_ref[1]
        cnt = cntp_ref[0, :, 0:1] + cntp_ref[1, :, 0:1]
        agg = jnp.dot(agg16, we_ref[...], preferred_element_type=jnp.float32)
        agg = agg + cnt * be_ref[...]
        x1 = x_ref[...] + agg
        xl = jnp.dot(x1, wc_ref[...], preferred_element_type=jnp.float32)
        dinv = lax.rsqrt(cnt + 1.0)
        y_ref[...] = dinv * xl
        dinv_ref[...] = dinv

    return pl.pallas_call(
        body,
        in_specs=[
            pl.BlockSpec(memory_space=pl.ANY) if i == 6 else pl.BlockSpec()
            for i in range(7)
        ],
        out_shape=(
            jax.ShapeDtypeStruct((N, D), jnp.float32),
            jax.ShapeDtypeStruct((N, 1), jnp.float32),
        ),
    )(x, aggp, cntp, W_edge, b_edge2, W_conv, edge_attr)


def _tc_dense2(sp, y, dinv, b_conv2):
    def body(sp_ref, y_ref, dinv_ref, bc_ref, out_ref):
        s = sp_ref[0] + sp_ref[1] + y_ref[...]
        out_ref[...] = jnp.maximum(dinv_ref[...] * s + bc_ref[...], 0.0)

    return pl.pallas_call(
        body,
        out_shape=jax.ShapeDtypeStruct((N, D), jnp.float32),
    )(sp, y, dinv, b_conv2)


def kernel(x, edge_index, edge_attr, W_edge, b_edge, W_conv, b_conv):
    col4a = edge_index[1].reshape(NW, NGA, GA, CA)
    row4 = edge_index[0].reshape(NW, NGC, GC, CC)
    col4 = edge_index[1].reshape(NW, NGC, GC, CC)
    zeros16 = jnp.zeros((CA, ED), jnp.float32)
    ones8 = jnp.ones((CA, CW), jnp.float32)
    zeros8 = jnp.zeros((CA, CW), jnp.float32)
    zeros128 = jnp.zeros((CC, D), jnp.float32)

    aggp, cntp = _sc_scatter_attr(edge_attr, col4a, zeros16, ones8, zeros8)
    y, dinv = _tc_dense1(x, aggp, cntp, W_edge, b_edge.reshape(1, D), W_conv,
                         edge_attr)
    sp = _sc_gather_scatter(y, row4, col4, zeros128)
    return _tc_dense2(sp, y, dinv, b_conv.reshape(1, D))


# speedup vs baseline: 1.1437x; 1.1437x over previous
import functools

import jax
import jax.numpy as jnp
from jax import lax
from jax.experimental import pallas as pl
from jax.experimental.pallas import tpu as pltpu
from jax.experimental.pallas import tpu_sc as plsc

N = 10000
E = 320000
D = 128
ED = 16
CW = 8

NC = 2
NS = 16
NW = NC * NS
EPW = E // NW

CA = 80
GA = 25
NGA = EPW // (CA * GA)
CR = CA * ED // D

CC = 40
GC = 50
NGC = EPW // (CC * GC)

RPS = 624
TAIL = N - NS * RPS


def _sc_mesh():
    return plsc.VectorSubcoreMesh(
        core_axis_name="c", subcore_axis_name="s", num_cores=NC, num_subcores=NS
    )


def _sc_scatter_attr(edge_attr, col4a, zeros16, ones8, zeros8):
    @functools.partial(
        pl.kernel,
        out_type=(
            jax.ShapeDtypeStruct((NC, N, ED), jnp.float32),
            jax.ShapeDtypeStruct((NC, N, CW), jnp.float32),
        ),
        mesh=_sc_mesh(),
        compiler_params=pltpu.CompilerParams(use_tc_tiling_on_sc=False),
        scratch_types=[
            pltpu.VMEM((GA, CA), jnp.int32),
            pltpu.VMEM((CR, D), jnp.float32),
            pltpu.VMEM((CR, D), jnp.float32),
            pltpu.VMEM((CA, ED), jnp.float32),
            pltpu.VMEM((CA, ED), jnp.float32),
            pltpu.VMEM((CA, CW), jnp.float32),
            pltpu.VMEM((CA, CW), jnp.float32),
            pltpu.VMEM_SHARED((N, ED), jnp.float32),
            pltpu.VMEM_SHARED((N, CW), jnp.float32),
            pltpu.SemaphoreType.DMA,
            pltpu.SemaphoreType.DMA,
            pltpu.SemaphoreType.DMA,
            pltpu.SemaphoreType.DMA,
            pltpu.SemaphoreType.DMA,
        ],
    )
    def k(ea, col4_h, zeros_h, ones_h, zeros8_h, aggp, cntp,
          colv, lbuf0, lbuf1, abuf0, abuf1, onesv, zb8, acc16, acccnt,
          lsem0, lsem1, ssem0, ssem1, osem):
        cid = lax.axis_index("c")
        sid = lax.axis_index("s")
        wid = cid * NS + sid

        pltpu.sync_copy(zeros_h, abuf0)
        pltpu.sync_copy(ones_h, onesv)
        pltpu.sync_copy(zeros8_h, zb8)

        r0 = sid * RPS

        @pl.loop(0, 15)
        def _(i):
            pltpu.sync_copy(abuf0.at[pl.ds(0, 40)], acc16.at[pl.ds(r0 + i * 40, 40)])
            pltpu.sync_copy(zb8.at[pl.ds(0, 40)], acccnt.at[pl.ds(r0 + i * 40, 40)])
        pltpu.sync_copy(abuf0.at[pl.ds(0, 24)], acc16.at[pl.ds(r0 + 600, 24)])
        pltpu.sync_copy(zb8.at[pl.ds(0, 24)], acccnt.at[pl.ds(r0 + 600, 24)])

        @pl.when(sid == NS - 1)
        def _():
            pltpu.sync_copy(abuf0.at[pl.ds(0, TAIL)], acc16.at[pl.ds(NS * RPS, TAIL)])
            pltpu.sync_copy(zb8.at[pl.ds(0, TAIL)], acccnt.at[pl.ds(NS * RPS, TAIL)])

        plsc.subcore_barrier()

        base = wid * (EPW * ED // D)

        def load(j, buf, sem):
            pltpu.async_copy(ea.at[pl.ds(base + j * CR, CR)], buf, sem)

        def wait_load(buf, sem):
            pltpu.make_async_copy(ea.at[pl.ds(0, CR)], buf, sem).wait()

        def repack(lbuf, sbuf):
            for r in range(CR):
                for c in range(8):
                    sbuf[r * 8 + c, :] = lbuf[r, pl.ds(c * ED, ED)]

        def scat(j, buf, sem):
            pltpu.async_copy(buf, acc16.at[colv.at[j]], sem, add=True)

        def wait_scat(buf, sem):
            pltpu.make_async_copy(buf, acc16.at[colv.at[0]], sem).wait()

        def ones_scat(j):
            pltpu.async_copy(onesv, acccnt.at[colv.at[j]], osem, add=True)

        def wait_ones():
            pltpu.make_async_copy(onesv, acccnt.at[colv.at[0]], osem).wait()

        for g in range(NGA):
            gbase = g * GA
            pltpu.sync_copy(col4_h.at[wid, g], colv)
            load(gbase + 0, lbuf0, lsem0)
            load(gbase + 1, lbuf1, lsem1)

            @pl.loop(0, GA // 2 - 1)
            def _(t):
                a = gbase + 2 * t
                wait_load(lbuf0, lsem0)
                repack(lbuf0, abuf0)
                scat(2 * t, abuf0, ssem0)
                ones_scat(2 * t)
                wait_load(lbuf1, lsem1)
                repack(lbuf1, abuf1)
                scat(2 * t + 1, abuf1, ssem1)
                ones_scat(2 * t + 1)
                load(a + 2, lbuf0, lsem0)
                load(a + 3, lbuf1, lsem1)
                wait_scat(abuf0, ssem0)
                wait_scat(abuf1, ssem1)
                wait_ones()
                wait_ones()

            wait_load(lbuf0, lsem0)
            repack(lbuf0, abuf0)
            scat(GA - 3, abuf0, ssem0)
            ones_scat(GA - 3)
            wait_load(lbuf1, lsem1)
            repack(lbuf1, abuf1)
            scat(GA - 2, abuf1, ssem1)
            ones_scat(GA - 2)
            load(gbase + GA - 1, lbuf0, lsem0)
            wait_scat(abuf0, ssem0)
            wait_load(lbuf0, lsem0)
            repack(lbuf0, abuf0)
            scat(GA - 1, abuf0, ssem0)
            ones_scat(GA - 1)
            wait_scat(abuf0, ssem0)
            wait_scat(abuf1, ssem1)
            wait_ones()
            wait_ones()
            wait_ones()

        plsc.subcore_barrier()

        @pl.loop(0, 15)
        def _(i):
            pltpu.sync_copy(acc16.at[pl.ds(r0 + i * 40, 40)], abuf0.at[pl.ds(0, 40)])
            pltpu.sync_copy(abuf0.at[pl.ds(0, 40)], aggp.at[cid, pl.ds(r0 + i * 40, 40)])
            pltpu.sync_copy(acccnt.at[pl.ds(r0 + i * 40, 40)], zb8.at[pl.ds(0, 40)])
            pltpu.sync_copy(zb8.at[pl.ds(0, 40)], cntp.at[cid, pl.ds(r0 + i * 40, 40)])
        pltpu.sync_copy(acc16.at[pl.ds(r0 + 600, 24)], abuf0.at[pl.ds(0, 24)])
        pltpu.sync_copy(abuf0.at[pl.ds(0, 24)], aggp.at[cid, pl.ds(r0 + 600, 24)])
        pltpu.sync_copy(acccnt.at[pl.ds(r0 + 600, 24)], zb8.at[pl.ds(0, 24)])
        pltpu.sync_copy(zb8.at[pl.ds(0, 24)], cntp.at[cid, pl.ds(r0 + 600, 24)])

        @pl.when(sid == NS - 1)
        def _():
            pltpu.sync_copy(acc16.at[pl.ds(NS * RPS, TAIL)], abuf0.at[pl.ds(0, TAIL)])
            pltpu.sync_copy(abuf0.at[pl.ds(0, TAIL)], aggp.at[cid, pl.ds(NS * RPS, TAIL)])
            pltpu.sync_copy(acccnt.at[pl.ds(NS * RPS, TAIL)], zb8.at[pl.ds(0, TAIL)])
            pltpu.sync_copy(zb8.at[pl.ds(0, TAIL)], cntp.at[cid, pl.ds(NS * RPS, TAIL)])

    return k(edge_attr, col4a, zeros16, ones8, zeros8)


def _sc_gather_scatter(y, row4, col4, zeros128):
    @functools.partial(
        pl.kernel,
        out_type=jax.ShapeDtypeStruct((NC, N, D), jnp.float32),
        mesh=_sc_mesh(),
        compiler_params=pltpu.CompilerParams(use_tc_tiling_on_sc=False),
        scratch_types=[
            pltpu.VMEM((GC, CC), jnp.int32),
            pltpu.VMEM((GC, CC), jnp.int32),
            pltpu.VMEM((CC, D), jnp.float32),
            pltpu.VMEM((CC, D), jnp.float32),
            pltpu.VMEM((CC, D), jnp.float32),
            pltpu.VMEM((CC, D), jnp.float32),
            pltpu.VMEM_SHARED((N, D), jnp.float32),
            pltpu.SemaphoreType.DMA,
            pltpu.SemaphoreType.DMA,
            pltpu.SemaphoreType.DMA,
            pltpu.SemaphoreType.DMA,
            pltpu.SemaphoreType.DMA,
            pltpu.SemaphoreType.DMA,
            pltpu.SemaphoreType.DMA,
            pltpu.SemaphoreType.DMA,
        ],
    )
    def k(y_h, row4_h, col4_h, zeros_h, sp, rowv, colv,
          gbuf0, gbuf1, gbuf2, gbuf3, acc,
          gsem0, gsem1, gsem2, gsem3, ssem0, ssem1, ssem2, ssem3):
        cid = lax.axis_index("c")
        sid = lax.axis_index("s")
        wid = cid * NS + sid

        pltpu.sync_copy(zeros_h, gbuf0)

        r0 = sid * RPS

        @pl.loop(0, 15)
        def _(i):
            pltpu.sync_copy(gbuf0.at[pl.ds(0, 40)], acc.at[pl.ds(r0 + i * 40, 40)])
        pltpu.sync_copy(gbuf0.at[pl.ds(0, 24)], acc.at[pl.ds(r0 + 600, 24)])

        @pl.when(sid == NS - 1)
        def _():
            pltpu.sync_copy(gbuf0.at[pl.ds(0, TAIL)], acc.at[pl.ds(NS * RPS, TAIL)])

        plsc.subcore_barrier()

        def gath(j, buf, sem):
            pltpu.async_copy(y_h.at[rowv.at[j]], buf, sem)

        def wait_gath(buf, sem):
            pltpu.make_async_copy(y_h.at[rowv.at[0]], buf, sem).wait()

        def scat(j, buf, sem):
            pltpu.async_copy(buf, acc.at[colv.at[j]], sem, add=True)

        def wait_scat(buf, sem):
            pltpu.make_async_copy(buf, acc.at[colv.at[0]], sem).wait()

        bufs = (gbuf0, gbuf1, gbuf2, gbuf3)
        gsems = (gsem0, gsem1, gsem2, gsem3)
        ssems = (ssem0, ssem1, ssem2, ssem3)

        for g in range(NGC):
            pltpu.sync_copy(row4_h.at[wid, g], rowv)
            pltpu.sync_copy(col4_h.at[wid, g], colv)
            for i in range(4):
                gath(i, bufs[i], gsems[i])

            @pl.loop(0, GC // 4 - 1)
            def _(q):
                for i in range(4):
                    wait_gath(bufs[i], gsems[i])
                    scat(4 * q + i, bufs[i], ssems[i])
                for i in range(4):
                    wait_scat(bufs[i], ssems[i])
                    gath(4 * q + 4 + i, bufs[i], gsems[i])

            for i in range(4):
                wait_gath(bufs[i], gsems[i])
                scat(GC - 6 + i, bufs[i], ssems[i])
            wait_scat(gbuf0, ssem0)
            gath(GC - 2, gbuf0, gsem0)
            wait_scat(gbuf1, ssem1)
            gath(GC - 1, gbuf1, gsem1)
            wait_scat(gbuf2, ssem2)
            wait_scat(gbuf3, ssem3)
            wait_gath(gbuf0, gsem0)
            scat(GC - 2, gbuf0, ssem0)
            wait_gath(gbuf1, gsem1)
            scat(GC - 1, gbuf1, ssem1)
            wait_scat(gbuf0, ssem0)
            wait_scat(gbuf1, ssem1)

        plsc.subcore_barrier()

        @pl.loop(0, 15)
        def _(i):
            pltpu.sync_copy(acc.at[pl.ds(r0 + i * 40, 40)], gbuf0.at[pl.ds(0, 40)])
            pltpu.sync_copy(gbuf0.at[pl.ds(0, 40)], sp.at[cid, pl.ds(r0 + i * 40, 40)])
        pltpu.sync_copy(acc.at[pl.ds(r0 + 600, 24)], gbuf0.at[pl.ds(0, 24)])
        pltpu.sync_copy(gbuf0.at[pl.ds(0, 24)], sp.at[cid, pl.ds(r0 + 600, 24)])

        @pl.when(sid == NS - 1)
        def _():
            pltpu.sync_copy(acc.at[pl.ds(NS * RPS, TAIL)], gbuf0.at[pl.ds(0, TAIL)])
            pltpu.sync_copy(gbuf0.at[pl.ds(0, TAIL)], sp.at[cid, pl.ds(NS * RPS, TAIL)])

    return k(y, row4, col4, zeros128)


def _tc_dense1(x, aggp, cntp, W_edge, b_edge2, W_conv):
    def body(x_ref, aggp_ref, cntp_ref, we_ref, be_ref, wc_ref, y_ref, dinv_ref):
        agg16 = aggp_ref[0] + aggp_ref[1]
        cnt = cntp_ref[0, :, 0:1] + cntp_ref[1, :, 0:1]
        agg = jnp.dot(agg16, we_ref[...], preferred_element_type=jnp.float32)
        agg = agg + cnt * be_ref[...]
        x1 = x_ref[...] + agg
        xl = jnp.dot(x1, wc_ref[...], preferred_element_type=jnp.float32)
        dinv = lax.rsqrt(cnt + 1.0)
        y_ref[...] = dinv * xl
        dinv_ref[...] = dinv

    return pl.pallas_call(
        body,
        out_shape=(
            jax.ShapeDtypeStruct((N, D), jnp.float32),
            jax.ShapeDtypeStruct((N, 1), jnp.float32),
        ),
    )(x, aggp, cntp, W_edge, b_edge2, W_conv)


def _tc_dense2(sp, y, dinv, b_conv2):
    def body(sp_ref, y_ref, dinv_ref, bc_ref, out_ref):
        s = sp_ref[0] + sp_ref[1] + y_ref[...]
        out_ref[...] = jnp.maximum(dinv_ref[...] * s + bc_ref[...], 0.0)

    return pl.pallas_call(
        body,
        out_shape=jax.ShapeDtypeStruct((N, D), jnp.float32),
    )(sp, y, dinv, b_conv2)


def kernel(x, edge_index, edge_attr, W_edge, b_edge, W_conv, b_conv):
    col4a = edge_index[1].reshape(NW, NGA, GA, CA)
    row4 = edge_index[0].reshape(NW, NGC, GC, CC)
    col4 = edge_index[1].reshape(NW, NGC, GC, CC)
    ea128 = edge_attr.reshape(E * ED // D, D)
    zeros16 = jnp.zeros((CA, ED), jnp.float32)
    ones8 = jnp.ones((CA, CW), jnp.float32)
    zeros8 = jnp.zeros((CA, CW), jnp.float32)
    zeros128 = jnp.zeros((CC, D), jnp.float32)

    aggp, cntp = _sc_scatter_attr(ea128, col4a, zeros16, ones8, zeros8)
    y, dinv = _tc_dense1(x, aggp, cntp, W_edge, b_edge.reshape(1, D), W_conv)
    sp = _sc_gather_scatter(y, row4, col4, zeros128)
    return _tc_dense2(sp, y, dinv, b_conv.reshape(1, D))


# speedup vs baseline: 1.2019x; 1.0509x over previous
import functools

import jax
import jax.numpy as jnp
from jax import lax
from jax.experimental import pallas as pl
from jax.experimental.pallas import tpu as pltpu
from jax.experimental.pallas import tpu_sc as plsc

N = 10000
E = 320000
D = 128
ED = 16
CW = 8

NC = 2
NS = 16
NW = NC * NS
EPW = E // NW

CA = 125
GA = 40
NGA = EPW // (CA * GA)

CC = 40
GC = 50
NGC = EPW // (CC * GC)

RPS = 624
TAIL = N - NS * RPS


def _sc_mesh():
    return plsc.VectorSubcoreMesh(
        core_axis_name="c", subcore_axis_name="s", num_cores=NC, num_subcores=NS
    )


def _sc_scatter_attr(edge_attr, col4a, zeros16, ones8, zeros8):
    @functools.partial(
        pl.kernel,
        out_type=(
            jax.ShapeDtypeStruct((NC, N, ED), jnp.float32),
            jax.ShapeDtypeStruct((NC, N, CW), jnp.float32),
        ),
        mesh=_sc_mesh(),
        compiler_params=pltpu.CompilerParams(use_tc_tiling_on_sc=False),
        scratch_types=[
            pltpu.VMEM((GA, CA), jnp.int32),
            pltpu.VMEM((CA, ED), jnp.float32),
            pltpu.VMEM((CA, ED), jnp.float32),
            pltpu.VMEM((CA, CW), jnp.float32),
            pltpu.VMEM((CA, CW), jnp.float32),
            pltpu.VMEM_SHARED((N, ED), jnp.float32),
            pltpu.VMEM_SHARED((N, CW), jnp.float32),
            pltpu.SemaphoreType.DMA,
            pltpu.SemaphoreType.DMA,
            pltpu.SemaphoreType.DMA,
            pltpu.SemaphoreType.DMA,
            pltpu.SemaphoreType.DMA,
        ],
    )
    def k(ea, col4_h, zeros_h, ones_h, zeros8_h, aggp, cntp,
          colv, abuf0, abuf1, onesv, zb8, acc16, acccnt,
          lsem0, lsem1, ssem0, ssem1, osem):
        cid = lax.axis_index("c")
        sid = lax.axis_index("s")
        wid = cid * NS + sid

        pltpu.sync_copy(zeros_h, abuf0)
        pltpu.sync_copy(ones_h, onesv)
        pltpu.sync_copy(zeros8_h, zb8)

        r0 = sid * RPS

        @pl.loop(0, 15)
        def _(i):
            pltpu.sync_copy(abuf0.at[pl.ds(0, 40)], acc16.at[pl.ds(r0 + i * 40, 40)])
            pltpu.sync_copy(zb8.at[pl.ds(0, 40)], acccnt.at[pl.ds(r0 + i * 40, 40)])
        pltpu.sync_copy(abuf0.at[pl.ds(0, 24)], acc16.at[pl.ds(r0 + 600, 24)])
        pltpu.sync_copy(zb8.at[pl.ds(0, 24)], acccnt.at[pl.ds(r0 + 600, 24)])

        @pl.when(sid == NS - 1)
        def _():
            pltpu.sync_copy(abuf0.at[pl.ds(0, TAIL)], acc16.at[pl.ds(NS * RPS, TAIL)])
            pltpu.sync_copy(zb8.at[pl.ds(0, TAIL)], acccnt.at[pl.ds(NS * RPS, TAIL)])

        plsc.subcore_barrier()

        base = wid * EPW

        def load(j, buf, sem):
            pltpu.async_copy(ea.at[pl.ds(base + j * CA, CA)], buf, sem)

        def wait_load(buf, sem):
            pltpu.make_async_copy(ea.at[pl.ds(0, CA)], buf, sem).wait()

        def scat(j, buf, sem):
            pltpu.async_copy(buf, acc16.at[colv.at[j]], sem, add=True)

        def wait_scat(buf, sem):
            pltpu.make_async_copy(buf, acc16.at[colv.at[0]], sem).wait()

        def ones_scat(j):
            pltpu.async_copy(onesv, acccnt.at[colv.at[j]], osem, add=True)

        def wait_ones():
            pltpu.make_async_copy(onesv, acccnt.at[colv.at[0]], osem).wait()

        for g in range(NGA):
            gbase = g * GA
            pltpu.sync_copy(col4_h.at[wid, g], colv)
            load(gbase + 0, abuf0, lsem0)
            load(gbase + 1, abuf1, lsem1)

            @pl.loop(0, GA // 2 - 1)
            def _(t):
                a = gbase + 2 * t
                wait_load(abuf0, lsem0)
                scat(2 * t, abuf0, ssem0)
                ones_scat(2 * t)
                wait_load(abuf1, lsem1)
                scat(2 * t + 1, abuf1, ssem1)
                ones_scat(2 * t + 1)
                wait_scat(abuf0, ssem0)
                load(a + 2, abuf0, lsem0)
                wait_scat(abuf1, ssem1)
                load(a + 3, abuf1, lsem1)
                wait_ones()
                wait_ones()

            wait_load(abuf0, lsem0)
            scat(GA - 2, abuf0, ssem0)
            ones_scat(GA - 2)
            wait_load(abuf1, lsem1)
            scat(GA - 1, abuf1, ssem1)
            ones_scat(GA - 1)
            wait_scat(abuf0, ssem0)
            wait_scat(abuf1, ssem1)
            wait_ones()
            wait_ones()

        plsc.subcore_barrier()

        @pl.loop(0, 15)
        def _(i):
            pltpu.sync_copy(acc16.at[pl.ds(r0 + i * 40, 40)], abuf0.at[pl.ds(0, 40)])
            pltpu.sync_copy(abuf0.at[pl.ds(0, 40)], aggp.at[cid, pl.ds(r0 + i * 40, 40)])
            pltpu.sync_copy(acccnt.at[pl.ds(r0 + i * 40, 40)], zb8.at[pl.ds(0, 40)])
            pltpu.sync_copy(zb8.at[pl.ds(0, 40)], cntp.at[cid, pl.ds(r0 + i * 40, 40)])
        pltpu.sync_copy(acc16.at[pl.ds(r0 + 600, 24)], abuf0.at[pl.ds(0, 24)])
        pltpu.sync_copy(abuf0.at[pl.ds(0, 24)], aggp.at[cid, pl.ds(r0 + 600, 24)])
        pltpu.sync_copy(acccnt.at[pl.ds(r0 + 600, 24)], zb8.at[pl.ds(0, 24)])
        pltpu.sync_copy(zb8.at[pl.ds(0, 24)], cntp.at[cid, pl.ds(r0 + 600, 24)])

        @pl.when(sid == NS - 1)
        def _():
            pltpu.sync_copy(acc16.at[pl.ds(NS * RPS, TAIL)], abuf0.at[pl.ds(0, TAIL)])
            pltpu.sync_copy(abuf0.at[pl.ds(0, TAIL)], aggp.at[cid, pl.ds(NS * RPS, TAIL)])
            pltpu.sync_copy(acccnt.at[pl.ds(NS * RPS, TAIL)], zb8.at[pl.ds(0, TAIL)])
            pltpu.sync_copy(zb8.at[pl.ds(0, TAIL)], cntp.at[cid, pl.ds(NS * RPS, TAIL)])

    return k(edge_attr, col4a, zeros16, ones8, zeros8)


def _sc_gather_scatter(y, row4, col4, zeros128):
    @functools.partial(
        pl.kernel,
        out_type=jax.ShapeDtypeStruct((NC, N, D), jnp.float32),
        mesh=_sc_mesh(),
        compiler_params=pltpu.CompilerParams(use_tc_tiling_on_sc=False),
        scratch_types=[
            pltpu.VMEM((GC, CC), jnp.int32),
            pltpu.VMEM((GC, CC), jnp.int32),
            pltpu.VMEM((CC, D), jnp.float32),
            pltpu.VMEM((CC, D), jnp.float32),
            pltpu.VMEM((CC, D), jnp.float32),
            pltpu.VMEM((CC, D), jnp.float32),
            pltpu.VMEM_SHARED((N, D), jnp.float32),
            pltpu.SemaphoreType.DMA,
            pltpu.SemaphoreType.DMA,
            pltpu.SemaphoreType.DMA,
            pltpu.SemaphoreType.DMA,
            pltpu.SemaphoreType.DMA,
            pltpu.SemaphoreType.DMA,
            pltpu.SemaphoreType.DMA,
            pltpu.SemaphoreType.DMA,
        ],
    )
    def k(y_h, row4_h, col4_h, zeros_h, sp, rowv, colv,
          gbuf0, gbuf1, gbuf2, gbuf3, acc,
          gsem0, gsem1, gsem2, gsem3, ssem0, ssem1, ssem2, ssem3):
        cid = lax.axis_index("c")
        sid = lax.axis_index("s")
        wid = cid * NS + sid

        pltpu.sync_copy(zeros_h, gbuf0)

        r0 = sid * RPS

        @pl.loop(0, 15)
        def _(i):
            pltpu.sync_copy(gbuf0.at[pl.ds(0, 40)], acc.at[pl.ds(r0 + i * 40, 40)])
        pltpu.sync_copy(gbuf0.at[pl.ds(0, 24)], acc.at[pl.ds(r0 + 600, 24)])

        @pl.when(sid == NS - 1)
        def _():
            pltpu.sync_copy(gbuf0.at[pl.ds(0, TAIL)], acc.at[pl.ds(NS * RPS, TAIL)])

        plsc.subcore_barrier()

        def gath(j, buf, sem):
            pltpu.async_copy(y_h.at[rowv.at[j]], buf, sem)

        def wait_gath(buf, sem):
            pltpu.make_async_copy(y_h.at[rowv.at[0]], buf, sem).wait()

        def scat(j, buf, sem):
            pltpu.async_copy(buf, acc.at[colv.at[j]], sem, add=True)

        def wait_scat(buf, sem):
            pltpu.make_async_copy(buf, acc.at[colv.at[0]], sem).wait()

        bufs = (gbuf0, gbuf1, gbuf2, gbuf3)
        gsems = (gsem0, gsem1, gsem2, gsem3)
        ssems = (ssem0, ssem1, ssem2, ssem3)

        for g in range(NGC):
            pltpu.sync_copy(row4_h.at[wid, g], rowv)
            pltpu.sync_copy(col4_h.at[wid, g], colv)
            for i in range(4):
                gath(i, bufs[i], gsems[i])

            @pl.loop(0, GC // 4 - 1)
            def _(q):
                for i in range(4):
                    wait_gath(bufs[i], gsems[i])
                    scat(4 * q + i, bufs[i], ssems[i])
                for i in range(4):
                    wait_scat(bufs[i], ssems[i])
                    gath(4 * q + 4 + i, bufs[i], gsems[i])

            for i in range(4):
                wait_gath(bufs[i], gsems[i])
                scat(GC - 6 + i, bufs[i], ssems[i])
            wait_scat(gbuf0, ssem0)
            gath(GC - 2, gbuf0, gsem0)
            wait_scat(gbuf1, ssem1)
            gath(GC - 1, gbuf1, gsem1)
            wait_scat(gbuf2, ssem2)
            wait_scat(gbuf3, ssem3)
            wait_gath(gbuf0, gsem0)
            scat(GC - 2, gbuf0, ssem0)
            wait_gath(gbuf1, gsem1)
            scat(GC - 1, gbuf1, ssem1)
            wait_scat(gbuf0, ssem0)
            wait_scat(gbuf1, ssem1)

        plsc.subcore_barrier()

        @pl.loop(0, 15)
        def _(i):
            pltpu.sync_copy(acc.at[pl.ds(r0 + i * 40, 40)], gbuf0.at[pl.ds(0, 40)])
            pltpu.sync_copy(gbuf0.at[pl.ds(0, 40)], sp.at[cid, pl.ds(r0 + i * 40, 40)])
        pltpu.sync_copy(acc.at[pl.ds(r0 + 600, 24)], gbuf0.at[pl.ds(0, 24)])
        pltpu.sync_copy(gbuf0.at[pl.ds(0, 24)], sp.at[cid, pl.ds(r0 + 600, 24)])

        @pl.when(sid == NS - 1)
        def _():
            pltpu.sync_copy(acc.at[pl.ds(NS * RPS, TAIL)], gbuf0.at[pl.ds(0, TAIL)])
            pltpu.sync_copy(gbuf0.at[pl.ds(0, TAIL)], sp.at[cid, pl.ds(NS * RPS, TAIL)])

    return k(y, row4, col4, zeros128)


def _tc_dense1(x, aggp, cntp, W_edge, b_edge2, W_conv):
    def body(x_ref, aggp_ref, cntp_ref, we_ref, be_ref, wc_ref, y_ref, dinv_ref):
        agg16 = aggp_ref[0] + aggp_ref[1]
        cnt = cntp_ref[0, :, 0:1] + cntp_ref[1, :, 0:1]
        agg = jnp.dot(agg16, we_ref[...], preferred_element_type=jnp.float32)
        agg = agg + cnt * be_ref[...]
        x1 = x_ref[...] + agg
        xl = jnp.dot(x1, wc_ref[...], preferred_element_type=jnp.float32)
        dinv = lax.rsqrt(cnt + 1.0)
        y_ref[...] = dinv * xl
        dinv_ref[...] = dinv

    return pl.pallas_call(
        body,
        out_shape=(
            jax.ShapeDtypeStruct((N, D), jnp.float32),
            jax.ShapeDtypeStruct((N, 1), jnp.float32),
        ),
    )(x, aggp, cntp, W_edge, b_edge2, W_conv)


def _tc_dense2(sp, y, dinv, b_conv2):
    def body(sp_ref, y_ref, dinv_ref, bc_ref, out_ref):
        s = sp_ref[0] + sp_ref[1] + y_ref[...]
        out_ref[...] = jnp.maximum(dinv_ref[...] * s + bc_ref[...], 0.0)

    return pl.pallas_call(
        body,
        out_shape=jax.ShapeDtypeStruct((N, D), jnp.float32),
    )(sp, y, dinv, b_conv2)


def kernel(x, edge_index, edge_attr, W_edge, b_edge, W_conv, b_conv):
    col4a = edge_index[1].reshape(NW, NGA, GA, CA)
    row4 = edge_index[0].reshape(NW, NGC, GC, CC)
    col4 = edge_index[1].reshape(NW, NGC, GC, CC)
    zeros16 = jnp.zeros((CA, ED), jnp.float32)
    ones8 = jnp.ones((CA, CW), jnp.float32)
    zeros8 = jnp.zeros((CA, CW), jnp.float32)
    zeros128 = jnp.zeros((CC, D), jnp.float32)

    aggp, cntp = _sc_scatter_attr(edge_attr, col4a, zeros16, ones8, zeros8)
    y, dinv = _tc_dense1(x, aggp, cntp, W_edge, b_edge.reshape(1, D), W_conv)
    sp = _sc_gather_scatter(y, row4, col4, zeros128)
    return _tc_dense2(sp, y, dinv, b_conv.reshape(1, D))
